# Initial kernel scaffold; baseline (speedup 1.0000x reference)
import jax
import jax.numpy as jnp
from jax.experimental import pallas as pl


def kernel(x, edge_index, W1, a_src1, a_dst1, b1, W2, a_src2, a_dst2, b2, conv_w, conv_b):
    raise NotImplementedError("write your pallas kernel here")


# speedup vs baseline: 1.0405x; 1.0405x over previous
import functools

import jax
import jax.numpy as jnp
from jax.experimental import pallas as pl

N = 10000
FM = 128
H = 8
E = 320000
OUT_CH = 128
N_CIRC = 504

_BLK = 1000


def _stage_a_body(x_ref, w_ref, asrc_ref, adst_ref, xw_ref, logits_ref):
    xw = jnp.dot(x_ref[...], w_ref[...], preferred_element_type=jnp.float32)
    xw_ref[...] = xw
    xw3 = xw.reshape(_BLK, H, FM)
    als = (xw3 * asrc_ref[...][None]).sum(-1)
    ald = (xw3 * adst_ref[...][None]).sum(-1)
    logits_ref[...] = jnp.concatenate([als, ald], axis=1)


def _stage_a(x, W, a_src, a_dst):
    grid = N // _BLK
    return pl.pallas_call(
        _stage_a_body,
        grid=(grid,),
        in_specs=[
            pl.BlockSpec((_BLK, FM), lambda i: (i, 0)),
            pl.BlockSpec((FM, H * FM), lambda i: (0, 0)),
            pl.BlockSpec((H, FM), lambda i: (0, 0)),
            pl.BlockSpec((H, FM), lambda i: (0, 0)),
        ],
        out_specs=[
            pl.BlockSpec((_BLK, H * FM), lambda i: (i, 0)),
            pl.BlockSpec((_BLK, 2 * H), lambda i: (i, 0)),
        ],
        out_shape=[
            jax.ShapeDtypeStruct((N, H * FM), jnp.float32),
            jax.ShapeDtypeStruct((N, 2 * H), jnp.float32),
        ],
    )(x, W, a_src, a_dst)


def _stage_c_body(acc_ref, den_ref, b_ref, out_ref):
    acc = acc_ref[...].reshape(_BLK, H, FM)
    den = den_ref[...][:, :, None] + 1e-16
    out = (acc / den).mean(axis=1) + b_ref[...][None, :]
    out_ref[...] = jnp.maximum(out, 0.0)


def _stage_c(acc, denom, b):
    grid = N // _BLK
    return pl.pallas_call(
        _stage_c_body,
        grid=(grid,),
        in_specs=[
            pl.BlockSpec((_BLK, H * FM), lambda i: (i, 0)),
            pl.BlockSpec((_BLK, H), lambda i: (i, 0)),
            pl.BlockSpec((FM,), lambda i: (0,)),
        ],
        out_specs=pl.BlockSpec((_BLK, FM), lambda i: (i, 0)),
        out_shape=jax.ShapeDtypeStruct((N, FM), jnp.float32),
    )(acc, denom, b)


def _stage_e1_body(x1_ref, x2_ref, wc1_ref, wc2_ref, cb_ref, xo_ref):
    xo = jnp.dot(x1_ref[...], wc1_ref[...], preferred_element_type=jnp.float32)
    xo += jnp.dot(x2_ref[...], wc2_ref[...], preferred_element_type=jnp.float32)
    xo_ref[...] = xo + cb_ref[...][None, :]


def _stage_e1(x1, x2, wc1, wc2, conv_b):
    grid = N // _BLK
    return pl.pallas_call(
        _stage_e1_body,
        grid=(grid,),
        in_specs=[
            pl.BlockSpec((_BLK, FM), lambda i: (i, 0)),
            pl.BlockSpec((_BLK, FM), lambda i: (i, 0)),
            pl.BlockSpec((FM, OUT_CH), lambda i: (0, 0)),
            pl.BlockSpec((FM, OUT_CH), lambda i: (0, 0)),
            pl.BlockSpec((OUT_CH,), lambda i: (0,)),
        ],
        out_specs=pl.BlockSpec((_BLK, OUT_CH), lambda i: (i, 0)),
        out_shape=jax.ShapeDtypeStruct((N, OUT_CH), jnp.float32),
    )(x1, x2, wc1, wc2, conv_b)


def _stage_e2_body(circ_ref, mirna_ref, p_ref):
    p_ref[...] = jax.lax.dot_general(
        circ_ref[...], mirna_ref[...],
        (((1,), (1,)), ((), ())),
        preferred_element_type=jnp.float32)


def _stage_e2(circ, mirna):
    return pl.pallas_call(
        _stage_e2_body,
        in_specs=[
            pl.BlockSpec((N_CIRC, OUT_CH), lambda: (0, 0)),
            pl.BlockSpec((N - N_CIRC, OUT_CH), lambda: (0, 0)),
        ],
        out_specs=pl.BlockSpec((N_CIRC, N - N_CIRC), lambda: (0, 0)),
        out_shape=jax.ShapeDtypeStruct((N_CIRC, N - N_CIRC), jnp.float32),
    )(circ, mirna)


def _edge_phase(xw, logits, src, dst):
    als = logits[:, :H]
    ald = logits[:, H:]
    alpha = jax.nn.leaky_relu(als[src] + ald[dst], negative_slope=0.2)
    ex = jnp.exp(alpha)
    denom = jax.ops.segment_sum(ex, dst, num_segments=N)
    xw3 = xw.reshape(N, H, FM)
    acc = jax.ops.segment_sum(xw3[src] * ex[:, :, None], dst, num_segments=N)
    return acc.reshape(N, H * FM), denom


def kernel(x, edge_index, W1, a_src1, a_dst1, b1, W2, a_src2, a_dst2, b2,
           conv_w, conv_b):
    loops = jnp.arange(N, dtype=edge_index.dtype)
    src = jnp.concatenate([edge_index[0], loops])
    dst = jnp.concatenate([edge_index[1], loops])

    xw1, logits1 = _stage_a(x, W1, a_src1, a_dst1)
    acc1, den1 = _edge_phase(xw1, logits1, src, dst)
    x1 = _stage_c(acc1, den1, b1)

    xw2, logits2 = _stage_a(x1, W2, a_src2, a_dst2)
    acc2, den2 = _edge_phase(xw2, logits2, src, dst)
    x2 = _stage_c(acc2, den2, b2)

    wc = conv_w.reshape(OUT_CH, 2 * FM).T
    xo = _stage_e1(x1, x2, wc[:FM], wc[FM:], conv_b)
    circ = xo[:N_CIRC]
    mirna = xo[N_CIRC:]
    p = _stage_e2(circ, mirna)
    return p, circ, mirna


# speedup vs baseline: 4.4224x; 4.2502x over previous
import jax
import jax.numpy as jnp
from jax import lax
from jax.experimental import pallas as pl
from jax.experimental.pallas import tpu as pltpu
from jax.experimental.pallas import tpu_sc as plsc

N = 10000
FM = 128
H = 8
E = 320000
OUT_CH = 128
N_CIRC = 504

_BLK = 1000

EP = 348160
E_TILE = EP // 16
E_W = EP // 32
E_BLK = 640
NR = 10256
ND = 10240
DUMP = 10240
EPC = EP // 8

_GDN = lax.GatherDimensionNumbers(
    offset_dims=(), collapsed_slice_dims=(0,), start_index_map=(0,))


def _bcast16(v, i):
    idx = jnp.full((16,), i, jnp.int32)
    return lax.gather(v, idx[:, None], _GDN, slice_sizes=(1,),
                      mode=lax.GatherScatterMode.PROMISE_IN_BOUNDS)


def _stage_a_body(x_ref, w_ref, asrc_ref, adst_ref, xw_ref, logits_ref):
    xw = jnp.dot(x_ref[...], w_ref[...], preferred_element_type=jnp.float32)
    xw3 = xw.reshape(_BLK, H, FM)
    xw_ref[...] = xw3.transpose(1, 0, 2)
    als = (xw3 * asrc_ref[...][None]).sum(-1)
    ald = (xw3 * adst_ref[...][None]).sum(-1)
    logits_ref[...] = jnp.concatenate(
        [als, ald, jnp.zeros((_BLK, FM - 2 * H), jnp.float32)], axis=1)


def _stage_a(x, W, a_src, a_dst):
    grid = N // _BLK
    return pl.pallas_call(
        _stage_a_body,
        grid=(grid,),
        in_specs=[
            pl.BlockSpec((_BLK, FM), lambda i: (i, 0)),
            pl.BlockSpec((FM, H * FM), lambda i: (0, 0)),
            pl.BlockSpec((H, FM), lambda i: (0, 0)),
            pl.BlockSpec((H, FM), lambda i: (0, 0)),
        ],
        out_specs=[
            pl.BlockSpec((H, _BLK, FM), lambda i: (0, i, 0)),
            pl.BlockSpec((_BLK, FM), lambda i: (i, 0)),
        ],
        out_shape=[
            jax.ShapeDtypeStruct((H, N, FM), jnp.float32),
            jax.ShapeDtypeStruct((N, FM), jnp.float32),
        ],
    )(x, W, a_src, a_dst)


def _stage_c_body(acc_ref, den_ref, b_ref, out_ref):
    d2 = den_ref[...]
    den = d2[0, :, :H] + d2[1, :, :H] + 1e-16
    acc = acc_ref[...]
    s = jnp.zeros((_BLK, FM), jnp.float32)
    for h in range(H):
        s += acc[h] / den[:, h][:, None]
    out = s * (1.0 / H) + b_ref[...][None, :]
    out_ref[...] = jnp.maximum(out, 0.0)


def _stage_c(acc, denom, b):
    grid = N // _BLK
    return pl.pallas_call(
        _stage_c_body,
        grid=(grid,),
        in_specs=[
            pl.BlockSpec((H, _BLK, FM), lambda i: (0, i, 0)),
            pl.BlockSpec((2, _BLK, FM), lambda i: (0, i, 0)),
            pl.BlockSpec((FM,), lambda i: (0,)),
        ],
        out_specs=pl.BlockSpec((_BLK, FM), lambda i: (i, 0)),
        out_shape=jax.ShapeDtypeStruct((N, FM), jnp.float32),
    )(acc, denom, b)


def _stage_e1_body(x1_ref, x2_ref, wc1_ref, wc2_ref, cb_ref, xo_ref):
    xo = jnp.dot(x1_ref[...], wc1_ref[...], preferred_element_type=jnp.float32)
    xo += jnp.dot(x2_ref[...], wc2_ref[...], preferred_element_type=jnp.float32)
    xo_ref[...] = xo + cb_ref[...][None, :]


def _stage_e1(x1, x2, wc1, wc2, conv_b):
    grid = N // _BLK
    return pl.pallas_call(
        _stage_e1_body,
        grid=(grid,),
        in_specs=[
            pl.BlockSpec((_BLK, FM), lambda i: (i, 0)),
            pl.BlockSpec((_BLK, FM), lambda i: (i, 0)),
            pl.BlockSpec((FM, OUT_CH), lambda i: (0, 0)),
            pl.BlockSpec((FM, OUT_CH), lambda i: (0, 0)),
            pl.BlockSpec((OUT_CH,), lambda i: (0,)),
        ],
        out_specs=pl.BlockSpec((_BLK, OUT_CH), lambda i: (i, 0)),
        out_shape=jax.ShapeDtypeStruct((N, OUT_CH), jnp.float32),
    )(x1, x2, wc1, wc2, conv_b)


def _stage_e2_body(circ_ref, mirna_ref, p_ref):
    p_ref[...] = jax.lax.dot_general(
        circ_ref[...], mirna_ref[...],
        (((1,), (1,)), ((), ())),
        preferred_element_type=jnp.float32)


def _stage_e2(circ, mirna):
    return pl.pallas_call(
        _stage_e2_body,
        in_specs=[
            pl.BlockSpec((N_CIRC, OUT_CH), lambda: (0, 0)),
            pl.BlockSpec((N - N_CIRC, OUT_CH), lambda: (0, 0)),
        ],
        out_specs=pl.BlockSpec((N_CIRC, N - N_CIRC), lambda: (0, 0)),
        out_shape=jax.ShapeDtypeStruct((N_CIRC, N - N_CIRC), jnp.float32),
    )(circ, mirna)


def _p0_body(lg_hbm, src_hbm, dst_hbm, z_hbm,
             den_hbm, ex_hbm,
             src_blk, dst_blk, lrow, lrow2, exmat, exc,
             den_sh, sem1, sem2):
    core = lax.axis_index("c")
    sid = lax.axis_index("s")
    lane = lax.iota(jnp.int32, 16)
    perm8 = (lane + 8) & 15

    z0 = sid * 640
    pltpu.sync_copy(z_hbm.at[pl.ds(z0, 640)], den_sh.at[pl.ds(z0, 640)])

    @pl.when(sid == 0)
    def _ztail():
        pltpu.sync_copy(z_hbm.at[pl.ds(ND, NR - ND)],
                        den_sh.at[pl.ds(ND, NR - ND)])

    pltpu.sync_copy(z_hbm.at[pl.ds(0, 16)], exmat)
    plsc.subcore_barrier()

    wbase = (core * 16 + sid) * E_W

    def p0_block(b, _):
        off = wbase + b * E_BLK
        pltpu.sync_copy(src_hbm.at[pl.ds(off, E_BLK)], src_blk)
        pltpu.sync_copy(dst_hbm.at[pl.ds(off, E_BLK)], dst_blk)

        def p0_batch(j, _):
            sl = pl.ds(j * 16, 16)
            g1 = pltpu.async_copy(lg_hbm.at[src_blk.at[sl]], lrow, sem1)
            g2 = pltpu.async_copy(lg_hbm.at[dst_blk.at[sl]], lrow2, sem2)
            g1.wait()
            g2.wait()

            def p0_edge(e, _):
                a = lrow[e, pl.ds(0, 16)]
                b2 = lax.gather(lrow2[e, pl.ds(0, 16)], perm8[:, None],
                                _GDN, slice_sizes=(1,),
                                mode=lax.GatherScatterMode.PROMISE_IN_BOUNDS)
                sv = a + b2
                alpha = jnp.where(sv >= 0, sv, 0.2 * sv)
                ev = jnp.exp(alpha)
                exmat[e, pl.ds(0, 16)] = ev
                ei = j * 16 + e
                exc[ei >> 3, pl.ds((ei & 7) * 16, 16)] = ev
                return 0

            lax.fori_loop(0, 16, p0_edge, 0)
            dstv = dst_blk[sl]
            pltpu.async_copy(exmat, den_sh.at[dstv], sem1,
                             add=True).wait()
            return 0

        lax.fori_loop(0, E_BLK // 16, p0_batch, 0)
        exoff = pl.multiple_of(off // 8, 8)
        pltpu.sync_copy(exc, ex_hbm.at[pl.ds(exoff, E_BLK // 8)])
        return 0

    lax.fori_loop(0, E_W // E_BLK, p0_block, 0)
    plsc.subcore_barrier()
    pltpu.sync_copy(den_sh.at[pl.ds(z0, 640)],
                    den_hbm.at[pl.ds(core * ND + z0, 640)])


def _hp_body(xwf_hbm, src_hbm, dst_hbm, ex_hbm, z_hbm,
             acc_hbm,
             src_blk, dst_blk, ex_blk, xrows, idxbuf,
             acc_sh, sem1):
    core = lax.axis_index("c")
    sid = lax.axis_index("s")

    z0 = sid * 640
    tbase = sid * E_TILE

    def head_pass(hl, _):
        h = core * 4 + hl
        pltpu.sync_copy(z_hbm.at[pl.ds(z0, 640)], acc_sh.at[pl.ds(z0, 640)])

        @pl.when(sid == 0)
        def _ztail():
            pltpu.sync_copy(z_hbm.at[pl.ds(ND, NR - ND)],
                            acc_sh.at[pl.ds(ND, NR - ND)])

        plsc.subcore_barrier()

        def hp_block(b, _):
            off = tbase + b * E_BLK
            pltpu.sync_copy(src_hbm.at[pl.ds(off, E_BLK)], src_blk)
            pltpu.sync_copy(dst_hbm.at[pl.ds(off, E_BLK)], dst_blk)
            exoff = pl.multiple_of(off // 8, 8)
            pltpu.sync_copy(ex_hbm.at[pl.ds(exoff, E_BLK // 8)], ex_blk)

            def hp_batch(j, _):
                sl = pl.ds(j * 16, 16)
                idxbuf[...] = src_blk[sl] + h * N
                pltpu.async_copy(xwf_hbm.at[idxbuf], xrows, sem1).wait()

                def hp_edge(e, _):
                    ei = j * 16 + e
                    rowv = ex_blk[ei >> 3, pl.ds((ei & 7) * 16, 16)]
                    exh = _bcast16(rowv, h)
                    for q in range(FM // 16):
                        xrows[e, pl.ds(q * 16, 16)] = (
                            xrows[e, pl.ds(q * 16, 16)] * exh)
                    return 0

                lax.fori_loop(0, 16, hp_edge, 0)
                dstv = dst_blk[sl]
                pltpu.async_copy(xrows, acc_sh.at[dstv], sem1,
                                 add=True).wait()
                return 0

            lax.fori_loop(0, E_BLK // 16, hp_batch, 0)
            return 0

        lax.fori_loop(0, E_TILE // E_BLK, hp_block, 0)
        plsc.subcore_barrier()
        pltpu.sync_copy(acc_sh.at[pl.ds(z0, 640)],
                        acc_hbm.at[pl.ds(h * ND + z0, 640)])
        plsc.subcore_barrier()
        return 0

    lax.fori_loop(0, 4, head_pass, 0)


def _edge_phase(xwh, logits, srcp, dstp, z1):
    mesh = plsc.VectorSubcoreMesh(core_axis_name="c", subcore_axis_name="s")
    xwf = xwh.reshape(H * N, FM)
    lg_pad = jnp.zeros((NR, FM), jnp.float32).at[:N].set(logits)

    p0 = pl.kernel(
        _p0_body,
        out_type=[
            jax.ShapeDtypeStruct((2 * ND, FM), jnp.float32),
            jax.ShapeDtypeStruct((EPC, FM), jnp.float32),
        ],
        mesh=mesh,
        scratch_types=[
            pltpu.VMEM((E_BLK,), jnp.int32),
            pltpu.VMEM((E_BLK,), jnp.int32),
            pltpu.VMEM((16, FM), jnp.float32),
            pltpu.VMEM((16, FM), jnp.float32),
            pltpu.VMEM((16, FM), jnp.float32),
            pltpu.VMEM((E_BLK // 8, FM), jnp.float32),
            pltpu.VMEM_SHARED((NR, FM), jnp.float32),
            pltpu.SemaphoreType.DMA,
            pltpu.SemaphoreType.DMA,
        ],
    )
    den2, ex = p0(lg_pad, srcp, dstp, z1)

    hp = pl.kernel(
        _hp_body,
        out_type=jax.ShapeDtypeStruct((H * ND, FM), jnp.float32),
        mesh=mesh,
        scratch_types=[
            pltpu.VMEM((E_BLK,), jnp.int32),
            pltpu.VMEM((E_BLK,), jnp.int32),
            pltpu.VMEM((E_BLK // 8, FM), jnp.float32),
            pltpu.VMEM((16, FM), jnp.float32),
            pltpu.VMEM((16,), jnp.int32),
            pltpu.VMEM_SHARED((NR, FM), jnp.float32),
            pltpu.SemaphoreType.DMA,
        ],
    )
    acc = hp(xwf, srcp, dstp, ex, z1)
    return (acc.reshape(H, ND, FM)[:, :N],
            den2.reshape(2, ND, FM)[:, :N])


def kernel(x, edge_index, W1, a_src1, a_dst1, b1, W2, a_src2, a_dst2, b2,
           conv_w, conv_b):
    loops = jnp.arange(N, dtype=edge_index.dtype)
    pad = jnp.full((EP - E - N,), DUMP, dtype=edge_index.dtype)
    srcp = jnp.concatenate([edge_index[0], loops, jnp.zeros_like(pad)])
    dstp = jnp.concatenate([edge_index[1], loops, pad])
    z1 = jnp.zeros((NR, FM), jnp.float32)

    xwh1, logits1 = _stage_a(x, W1, a_src1, a_dst1)
    acc1, den1 = _edge_phase(xwh1, logits1, srcp, dstp, z1)
    x1 = _stage_c(acc1, den1, b1)

    xwh2, logits2 = _stage_a(x1, W2, a_src2, a_dst2)
    acc2, den2 = _edge_phase(xwh2, logits2, srcp, dstp, z1)
    x2 = _stage_c(acc2, den2, b2)

    wc = conv_w.reshape(OUT_CH, 2 * FM).T
    xo = _stage_e1(x1, x2, wc[:FM], wc[FM:], conv_b)
    circ = xo[:N_CIRC]
    mirna = xo[N_CIRC:]
    p = _stage_e2(circ, mirna)
    return p, circ, mirna


# speedup vs baseline: 5.4908x; 1.2416x over previous
import jax
import jax.numpy as jnp
from jax import lax
from jax.experimental import pallas as pl
from jax.experimental.pallas import tpu as pltpu
from jax.experimental.pallas import tpu_sc as plsc

N = 10000
FM = 128
H = 8
E = 320000
OUT_CH = 128
N_CIRC = 504

_BLK = 1000

EP = 348160
E_TILE = EP // 16
E_W = EP // 32
E_BLK = 640
NR = 10256
ND = 10240
DUMP = 10240
EPC = EP // 8

_GDN = lax.GatherDimensionNumbers(
    offset_dims=(), collapsed_slice_dims=(0,), start_index_map=(0,))


def _bcast16(v, i):
    idx = jnp.full((16,), i, jnp.int32)
    return lax.gather(v, idx[:, None], _GDN, slice_sizes=(1,),
                      mode=lax.GatherScatterMode.PROMISE_IN_BOUNDS)


def _stage_a_body(x_ref, w_ref, asrc_ref, adst_ref, xw_ref, logits_ref):
    xw = jnp.dot(x_ref[...], w_ref[...], preferred_element_type=jnp.float32)
    xw3 = xw.reshape(_BLK, H, FM)
    xw_ref[...] = xw3.transpose(1, 0, 2)
    als = (xw3 * asrc_ref[...][None]).sum(-1)
    ald = (xw3 * adst_ref[...][None]).sum(-1)
    logits_ref[...] = jnp.concatenate(
        [als, ald, jnp.zeros((_BLK, FM - 2 * H), jnp.float32)], axis=1)


def _stage_a(x, W, a_src, a_dst):
    grid = N // _BLK
    return pl.pallas_call(
        _stage_a_body,
        grid=(grid,),
        in_specs=[
            pl.BlockSpec((_BLK, FM), lambda i: (i, 0)),
            pl.BlockSpec((FM, H * FM), lambda i: (0, 0)),
            pl.BlockSpec((H, FM), lambda i: (0, 0)),
            pl.BlockSpec((H, FM), lambda i: (0, 0)),
        ],
        out_specs=[
            pl.BlockSpec((H, _BLK, FM), lambda i: (0, i, 0)),
            pl.BlockSpec((_BLK, FM), lambda i: (i, 0)),
        ],
        out_shape=[
            jax.ShapeDtypeStruct((H, N, FM), jnp.float32),
            jax.ShapeDtypeStruct((N, FM), jnp.float32),
        ],
    )(x, W, a_src, a_dst)


def _stage_c_body(acc_ref, den_ref, b_ref, out_ref):
    d2 = den_ref[...]
    den = d2[0, :, :H] + d2[1, :, :H] + 1e-16
    acc = acc_ref[...]
    s = jnp.zeros((_BLK, FM), jnp.float32)
    for h in range(H):
        s += acc[h] / den[:, h][:, None]
    out = s * (1.0 / H) + b_ref[...][None, :]
    out_ref[...] = jnp.maximum(out, 0.0)


def _stage_c(acc, denom, b):
    grid = N // _BLK
    return pl.pallas_call(
        _stage_c_body,
        grid=(grid,),
        in_specs=[
            pl.BlockSpec((H, _BLK, FM), lambda i: (0, i, 0)),
            pl.BlockSpec((2, _BLK, FM), lambda i: (0, i, 0)),
            pl.BlockSpec((FM,), lambda i: (0,)),
        ],
        out_specs=pl.BlockSpec((_BLK, FM), lambda i: (i, 0)),
        out_shape=jax.ShapeDtypeStruct((N, FM), jnp.float32),
    )(acc, denom, b)


def _stage_e1_body(x1_ref, x2_ref, wc1_ref, wc2_ref, cb_ref, xo_ref):
    xo = jnp.dot(x1_ref[...], wc1_ref[...], preferred_element_type=jnp.float32)
    xo += jnp.dot(x2_ref[...], wc2_ref[...], preferred_element_type=jnp.float32)
    xo_ref[...] = xo + cb_ref[...][None, :]


def _stage_e1(x1, x2, wc1, wc2, conv_b):
    grid = N // _BLK
    return pl.pallas_call(
        _stage_e1_body,
        grid=(grid,),
        in_specs=[
            pl.BlockSpec((_BLK, FM), lambda i: (i, 0)),
            pl.BlockSpec((_BLK, FM), lambda i: (i, 0)),
            pl.BlockSpec((FM, OUT_CH), lambda i: (0, 0)),
            pl.BlockSpec((FM, OUT_CH), lambda i: (0, 0)),
            pl.BlockSpec((OUT_CH,), lambda i: (0,)),
        ],
        out_specs=pl.BlockSpec((_BLK, OUT_CH), lambda i: (i, 0)),
        out_shape=jax.ShapeDtypeStruct((N, OUT_CH), jnp.float32),
    )(x1, x2, wc1, wc2, conv_b)


def _stage_e2_body(circ_ref, mirna_ref, p_ref):
    p_ref[...] = jax.lax.dot_general(
        circ_ref[...], mirna_ref[...],
        (((1,), (1,)), ((), ())),
        preferred_element_type=jnp.float32)


def _stage_e2(circ, mirna):
    return pl.pallas_call(
        _stage_e2_body,
        in_specs=[
            pl.BlockSpec((N_CIRC, OUT_CH), lambda: (0, 0)),
            pl.BlockSpec((N - N_CIRC, OUT_CH), lambda: (0, 0)),
        ],
        out_specs=pl.BlockSpec((N_CIRC, N - N_CIRC), lambda: (0, 0)),
        out_shape=jax.ShapeDtypeStruct((N_CIRC, N - N_CIRC), jnp.float32),
    )(circ, mirna)


def _p0_body(lg_hbm, src_hbm, dst_hbm, z_hbm,
             den_hbm, ex_hbm,
             src_blk, dst_blk, lrow, lrow2, exmat, exc,
             den_sh, sem1, sem2):
    core = lax.axis_index("c")
    sid = lax.axis_index("s")
    lane = lax.iota(jnp.int32, 16)
    perm8 = (lane + 8) & 15

    z0 = sid * 640
    pltpu.sync_copy(z_hbm.at[pl.ds(z0, 640)], den_sh.at[pl.ds(z0, 640)])

    @pl.when(sid == 0)
    def _ztail():
        pltpu.sync_copy(z_hbm.at[pl.ds(ND, NR - ND)],
                        den_sh.at[pl.ds(ND, NR - ND)])

    pltpu.sync_copy(z_hbm.at[pl.ds(0, 16)], exmat)
    plsc.subcore_barrier()

    wbase = (core * 16 + sid) * E_W

    def p0_block(b, _):
        off = wbase + b * E_BLK
        pltpu.sync_copy(src_hbm.at[pl.ds(off, E_BLK)], src_blk)
        pltpu.sync_copy(dst_hbm.at[pl.ds(off, E_BLK)], dst_blk)

        def p0_batch(j, _):
            sl = pl.ds(j * 16, 16)
            g1 = pltpu.async_copy(lg_hbm.at[src_blk.at[sl]], lrow, sem1)
            g2 = pltpu.async_copy(lg_hbm.at[dst_blk.at[sl]], lrow2, sem2)
            g1.wait()
            g2.wait()

            def p0_edge(e, _):
                a = lrow[e, pl.ds(0, 16)]
                b2 = lax.gather(lrow2[e, pl.ds(0, 16)], perm8[:, None],
                                _GDN, slice_sizes=(1,),
                                mode=lax.GatherScatterMode.PROMISE_IN_BOUNDS)
                sv = a + b2
                alpha = jnp.where(sv >= 0, sv, 0.2 * sv)
                ev = jnp.exp(alpha)
                exmat[e, pl.ds(0, 16)] = ev
                ei = j * 16 + e
                exc[ei >> 3, pl.ds((ei & 7) * 16, 16)] = ev
                return 0

            lax.fori_loop(0, 16, p0_edge, 0)
            dstv = dst_blk[sl]
            pltpu.async_copy(exmat, den_sh.at[dstv], sem1,
                             add=True).wait()
            return 0

        lax.fori_loop(0, E_BLK // 16, p0_batch, 0)
        exoff = pl.multiple_of(off // 8, 8)
        pltpu.sync_copy(exc, ex_hbm.at[pl.ds(exoff, E_BLK // 8)])
        return 0

    lax.fori_loop(0, E_W // E_BLK, p0_block, 0)
    plsc.subcore_barrier()
    pltpu.sync_copy(den_sh.at[pl.ds(z0, 640)],
                    den_hbm.at[pl.ds(core * ND + z0, 640)])


def _hp_body(xwf_hbm, src_hbm, dst_hbm, ex_hbm, z_hbm,
             acc_hbm,
             src_blk, dst_blk, ex_blk, xrows, idxbuf, dstbuf,
             acc_sh, sem1):
    core = lax.axis_index("c")
    sid = lax.axis_index("s")

    z0 = sid * 640
    tbase = sid * E_TILE

    def head_pass(hl, _):
        h = core * 4 + hl
        pltpu.sync_copy(z_hbm.at[pl.ds(z0, 640)], acc_sh.at[pl.ds(z0, 640)])

        @pl.when(sid == 0)
        def _ztail():
            pltpu.sync_copy(z_hbm.at[pl.ds(ND, NR - ND)],
                            acc_sh.at[pl.ds(ND, NR - ND)])

        plsc.subcore_barrier()

        def hp_block(b, _):
            off = tbase + b * E_BLK
            pltpu.sync_copy(src_hbm.at[pl.ds(off, E_BLK)], src_blk)
            pltpu.sync_copy(dst_hbm.at[pl.ds(off, E_BLK)], dst_blk)
            exoff = pl.multiple_of(off // 8, 8)
            pltpu.sync_copy(ex_hbm.at[pl.ds(exoff, E_BLK // 8)], ex_blk)

            def hp_batch(j, _):
                for k in range(4):
                    slk = pl.ds(j * 64 + k * 16, 16)
                    idxbuf[pl.ds(k * 16, 16)] = src_blk[slk] + h * N
                    dstbuf[pl.ds(k * 16, 16)] = dst_blk[slk]
                pltpu.async_copy(xwf_hbm.at[idxbuf], xrows, sem1).wait()

                def hp_edge(e, _):
                    ei = j * 64 + e
                    rowv = ex_blk[ei >> 3, pl.ds((ei & 7) * 16, 16)]
                    exh = _bcast16(rowv, h)
                    for q in range(FM // 16):
                        xrows[e, pl.ds(q * 16, 16)] = (
                            xrows[e, pl.ds(q * 16, 16)] * exh)
                    return 0

                lax.fori_loop(0, 64, hp_edge, 0)
                pltpu.async_copy(xrows, acc_sh.at[dstbuf], sem1,
                                 add=True).wait()
                return 0

            lax.fori_loop(0, E_BLK // 64, hp_batch, 0)
            return 0

        lax.fori_loop(0, E_TILE // E_BLK, hp_block, 0)
        plsc.subcore_barrier()
        pltpu.sync_copy(acc_sh.at[pl.ds(z0, 640)],
                        acc_hbm.at[pl.ds(h * ND + z0, 640)])
        plsc.subcore_barrier()
        return 0

    lax.fori_loop(0, 4, head_pass, 0)


def _edge_phase(xwh, logits, srcp, dstp, z1):
    mesh = plsc.VectorSubcoreMesh(core_axis_name="c", subcore_axis_name="s")
    xwf = xwh.reshape(H * N, FM)
    lg_pad = jnp.zeros((NR, FM), jnp.float32).at[:N].set(logits)

    p0 = pl.kernel(
        _p0_body,
        out_type=[
            jax.ShapeDtypeStruct((2 * ND, FM), jnp.float32),
            jax.ShapeDtypeStruct((EPC, FM), jnp.float32),
        ],
        mesh=mesh,
        scratch_types=[
            pltpu.VMEM((E_BLK,), jnp.int32),
            pltpu.VMEM((E_BLK,), jnp.int32),
            pltpu.VMEM((16, FM), jnp.float32),
            pltpu.VMEM((16, FM), jnp.float32),
            pltpu.VMEM((16, FM), jnp.float32),
            pltpu.VMEM((E_BLK // 8, FM), jnp.float32),
            pltpu.VMEM_SHARED((NR, FM), jnp.float32),
            pltpu.SemaphoreType.DMA,
            pltpu.SemaphoreType.DMA,
        ],
    )
    den2, ex = p0(lg_pad, srcp, dstp, z1)

    hp = pl.kernel(
        _hp_body,
        out_type=jax.ShapeDtypeStruct((H * ND, FM), jnp.float32),
        mesh=mesh,
        scratch_types=[
            pltpu.VMEM((E_BLK,), jnp.int32),
            pltpu.VMEM((E_BLK,), jnp.int32),
            pltpu.VMEM((E_BLK // 8, FM), jnp.float32),
            pltpu.VMEM((64, FM), jnp.float32),
            pltpu.VMEM((64,), jnp.int32),
            pltpu.VMEM((64,), jnp.int32),
            pltpu.VMEM_SHARED((NR, FM), jnp.float32),
            pltpu.SemaphoreType.DMA,
        ],
    )
    acc = hp(xwf, srcp, dstp, ex, z1)
    return (acc.reshape(H, ND, FM)[:, :N],
            den2.reshape(2, ND, FM)[:, :N])


def kernel(x, edge_index, W1, a_src1, a_dst1, b1, W2, a_src2, a_dst2, b2,
           conv_w, conv_b):
    loops = jnp.arange(N, dtype=edge_index.dtype)
    pad = jnp.full((EP - E - N,), DUMP, dtype=edge_index.dtype)
    srcp = jnp.concatenate([edge_index[0], loops, jnp.zeros_like(pad)])
    dstp = jnp.concatenate([edge_index[1], loops, pad])
    z1 = jnp.zeros((NR, FM), jnp.float32)

    xwh1, logits1 = _stage_a(x, W1, a_src1, a_dst1)
    acc1, den1 = _edge_phase(xwh1, logits1, srcp, dstp, z1)
    x1 = _stage_c(acc1, den1, b1)

    xwh2, logits2 = _stage_a(x1, W2, a_src2, a_dst2)
    acc2, den2 = _edge_phase(xwh2, logits2, srcp, dstp, z1)
    x2 = _stage_c(acc2, den2, b2)

    wc = conv_w.reshape(OUT_CH, 2 * FM).T
    xo = _stage_e1(x1, x2, wc[:FM], wc[FM:], conv_b)
    circ = xo[:N_CIRC]
    mirna = xo[N_CIRC:]
    p = _stage_e2(circ, mirna)
    return p, circ, mirna


# speedup vs baseline: 6.5385x; 1.1908x over previous
import jax
import jax.numpy as jnp
from jax import lax
from jax.experimental import pallas as pl
from jax.experimental.pallas import tpu as pltpu
from jax.experimental.pallas import tpu_sc as plsc

N = 10000
FM = 128
H = 8
E = 320000
OUT_CH = 128
N_CIRC = 504

_BLK = 1000

EP = 348160
E_TILE = EP // 16
E_W = EP // 32
E_BLK = 640
NR = 10256
ND = 10240
DUMP = 10240
EPC = EP // 8

_GDN = lax.GatherDimensionNumbers(
    offset_dims=(), collapsed_slice_dims=(0,), start_index_map=(0,))


def _bcast16(v, i):
    idx = jnp.full((16,), i, jnp.int32)
    return lax.gather(v, idx[:, None], _GDN, slice_sizes=(1,),
                      mode=lax.GatherScatterMode.PROMISE_IN_BOUNDS)


def _stage_a_body(x_ref, w_ref, asrc_ref, adst_ref, xw_ref, logits_ref):
    xw = jnp.dot(x_ref[...], w_ref[...], preferred_element_type=jnp.float32)
    xw3 = xw.reshape(_BLK, H, FM)
    xw_ref[...] = xw3.transpose(1, 0, 2)
    als = (xw3 * asrc_ref[...][None]).sum(-1)
    ald = (xw3 * adst_ref[...][None]).sum(-1)
    logits_ref[...] = jnp.concatenate(
        [als, ald, jnp.zeros((_BLK, FM - 2 * H), jnp.float32)], axis=1)


def _stage_a(x, W, a_src, a_dst):
    grid = N // _BLK
    return pl.pallas_call(
        _stage_a_body,
        grid=(grid,),
        in_specs=[
            pl.BlockSpec((_BLK, FM), lambda i: (i, 0)),
            pl.BlockSpec((FM, H * FM), lambda i: (0, 0)),
            pl.BlockSpec((H, FM), lambda i: (0, 0)),
            pl.BlockSpec((H, FM), lambda i: (0, 0)),
        ],
        out_specs=[
            pl.BlockSpec((H, _BLK, FM), lambda i: (0, i, 0)),
            pl.BlockSpec((_BLK, FM), lambda i: (i, 0)),
        ],
        out_shape=[
            jax.ShapeDtypeStruct((H, N, FM), jnp.float32),
            jax.ShapeDtypeStruct((N, FM), jnp.float32),
        ],
    )(x, W, a_src, a_dst)


def _stage_c_body(acc_ref, den_ref, b_ref, out_ref):
    d2 = den_ref[...]
    den = d2[0, :, :H] + d2[1, :, :H] + 1e-16
    acc = acc_ref[...]
    s = jnp.zeros((_BLK, FM), jnp.float32)
    for h in range(H):
        s += acc[h] / den[:, h][:, None]
    out = s * (1.0 / H) + b_ref[...][None, :]
    out_ref[...] = jnp.maximum(out, 0.0)


def _stage_c(acc, denom, b):
    grid = N // _BLK
    return pl.pallas_call(
        _stage_c_body,
        grid=(grid,),
        in_specs=[
            pl.BlockSpec((H, _BLK, FM), lambda i: (0, i, 0)),
            pl.BlockSpec((2, _BLK, FM), lambda i: (0, i, 0)),
            pl.BlockSpec((FM,), lambda i: (0,)),
        ],
        out_specs=pl.BlockSpec((_BLK, FM), lambda i: (i, 0)),
        out_shape=jax.ShapeDtypeStruct((N, FM), jnp.float32),
    )(acc, denom, b)


def _stage_e1_body(x1_ref, x2_ref, wc1_ref, wc2_ref, cb_ref, xo_ref):
    xo = jnp.dot(x1_ref[...], wc1_ref[...], preferred_element_type=jnp.float32)
    xo += jnp.dot(x2_ref[...], wc2_ref[...], preferred_element_type=jnp.float32)
    xo_ref[...] = xo + cb_ref[...][None, :]


def _stage_e1(x1, x2, wc1, wc2, conv_b):
    grid = N // _BLK
    return pl.pallas_call(
        _stage_e1_body,
        grid=(grid,),
        in_specs=[
            pl.BlockSpec((_BLK, FM), lambda i: (i, 0)),
            pl.BlockSpec((_BLK, FM), lambda i: (i, 0)),
            pl.BlockSpec((FM, OUT_CH), lambda i: (0, 0)),
            pl.BlockSpec((FM, OUT_CH), lambda i: (0, 0)),
            pl.BlockSpec((OUT_CH,), lambda i: (0,)),
        ],
        out_specs=pl.BlockSpec((_BLK, OUT_CH), lambda i: (i, 0)),
        out_shape=jax.ShapeDtypeStruct((N, OUT_CH), jnp.float32),
    )(x1, x2, wc1, wc2, conv_b)


def _stage_e2_body(circ_ref, mirna_ref, p_ref):
    p_ref[...] = jax.lax.dot_general(
        circ_ref[...], mirna_ref[...],
        (((1,), (1,)), ((), ())),
        preferred_element_type=jnp.float32)


def _stage_e2(circ, mirna):
    return pl.pallas_call(
        _stage_e2_body,
        in_specs=[
            pl.BlockSpec((N_CIRC, OUT_CH), lambda: (0, 0)),
            pl.BlockSpec((N - N_CIRC, OUT_CH), lambda: (0, 0)),
        ],
        out_specs=pl.BlockSpec((N_CIRC, N - N_CIRC), lambda: (0, 0)),
        out_shape=jax.ShapeDtypeStruct((N_CIRC, N - N_CIRC), jnp.float32),
    )(circ, mirna)


def _p0_body(lg_hbm, src_hbm, dst_hbm, z_hbm,
             den_hbm, ex_hbm,
             src_blk, dst_blk, lrow, lrow2, exmat, exc,
             den_sh, sem1, sem2):
    core = lax.axis_index("c")
    sid = lax.axis_index("s")
    lane = lax.iota(jnp.int32, 16)
    perm8 = (lane + 8) & 15

    z0 = sid * 640
    pltpu.sync_copy(z_hbm.at[pl.ds(z0, 640)], den_sh.at[pl.ds(z0, 640)])

    @pl.when(sid == 0)
    def _ztail():
        pltpu.sync_copy(z_hbm.at[pl.ds(ND, NR - ND)],
                        den_sh.at[pl.ds(ND, NR - ND)])

    pltpu.sync_copy(z_hbm.at[pl.ds(0, 16)], exmat)
    plsc.subcore_barrier()

    wbase = (core * 16 + sid) * E_W

    def p0_block(b, _):
        off = wbase + b * E_BLK
        pltpu.sync_copy(src_hbm.at[pl.ds(off, E_BLK)], src_blk)
        pltpu.sync_copy(dst_hbm.at[pl.ds(off, E_BLK)], dst_blk)

        def p0_batch(j, _):
            sl = pl.ds(j * 16, 16)
            g1 = pltpu.async_copy(lg_hbm.at[src_blk.at[sl]], lrow, sem1)
            g2 = pltpu.async_copy(lg_hbm.at[dst_blk.at[sl]], lrow2, sem2)
            g1.wait()
            g2.wait()

            def p0_edge(e, _):
                a = lrow[e, pl.ds(0, 16)]
                b2 = lax.gather(lrow2[e, pl.ds(0, 16)], perm8[:, None],
                                _GDN, slice_sizes=(1,),
                                mode=lax.GatherScatterMode.PROMISE_IN_BOUNDS)
                sv = a + b2
                alpha = jnp.where(sv >= 0, sv, 0.2 * sv)
                ev = jnp.exp(alpha)
                exmat[e, pl.ds(0, 16)] = ev
                ei = j * 16 + e
                exc[ei >> 3, pl.ds((ei & 7) * 16, 16)] = ev
                return 0

            lax.fori_loop(0, 16, p0_edge, 0)
            dstv = dst_blk[sl]
            pltpu.async_copy(exmat, den_sh.at[dstv], sem1,
                             add=True).wait()
            return 0

        lax.fori_loop(0, E_BLK // 16, p0_batch, 0)
        exoff = pl.multiple_of(off // 8, 8)
        pltpu.sync_copy(exc, ex_hbm.at[pl.ds(exoff, E_BLK // 8)])
        return 0

    lax.fori_loop(0, E_W // E_BLK, p0_block, 0)
    plsc.subcore_barrier()
    pltpu.sync_copy(den_sh.at[pl.ds(z0, 640)],
                    den_hbm.at[pl.ds(core * ND + z0, 640)])


def _hp_body(xwf_hbm, src_hbm, dst_hbm, ex_hbm, z_hbm,
             acc_hbm,
             src_blk, dst_blk, ex_blk, xr0, xr1, idxb, dstb,
             acc_sh, sem1, sem2):
    core = lax.axis_index("c")
    sid = lax.axis_index("s")

    z0 = sid * 640
    tbase = sid * E_TILE
    NB = E_BLK // 64

    def head_pass(hl, _):
        h = core * 4 + hl
        pltpu.sync_copy(z_hbm.at[pl.ds(z0, 640)], acc_sh.at[pl.ds(z0, 640)])

        @pl.when(sid == 0)
        def _ztail():
            pltpu.sync_copy(z_hbm.at[pl.ds(ND, NR - ND)],
                            acc_sh.at[pl.ds(ND, NR - ND)])

        plsc.subcore_barrier()

        def hp_block(b, _):
            off = tbase + b * E_BLK
            pltpu.sync_copy(src_hbm.at[pl.ds(off, E_BLK)], src_blk)
            pltpu.sync_copy(dst_hbm.at[pl.ds(off, E_BLK)], dst_blk)
            exoff = pl.multiple_of(off // 8, 8)
            pltpu.sync_copy(ex_hbm.at[pl.ds(exoff, E_BLK // 8)], ex_blk)

            xr = (xr0, xr1)

            def fill(j):
                p = j & 1
                for k in range(4):
                    slk = pl.ds(j * 64 + k * 16, 16)
                    idxb[p, pl.ds(k * 16, 16)] = src_blk[slk] + h * N
                    dstb[p, pl.ds(k * 16, 16)] = dst_blk[slk]

            fill(0)
            g = pltpu.async_copy(xwf_hbm.at[idxb.at[0]], xr0, sem1)
            prev_sc = None
            for j in range(NB):
                p = j & 1
                g.wait()
                if prev_sc is not None:
                    prev_sc.wait()
                if j + 1 < NB:
                    fill(j + 1)
                    g = pltpu.async_copy(xwf_hbm.at[idxb.at[1 - p]],
                                         xr[1 - p], sem1)

                xrp = xr[p]

                def hp_edge(e, _, _j=j, _xrp=xrp):
                    rowv = ex_blk[_j * 8 + (e >> 3), pl.ds((e & 7) * 16, 16)]
                    exh = _bcast16(rowv, h)
                    for q in range(FM // 16):
                        _xrp[e, pl.ds(q * 16, 16)] = (
                            _xrp[e, pl.ds(q * 16, 16)] * exh)
                    return 0

                lax.fori_loop(0, 64, hp_edge, 0)
                prev_sc = pltpu.async_copy(xrp, acc_sh.at[dstb.at[p]], sem2,
                                           add=True)
            prev_sc.wait()
            return 0

        lax.fori_loop(0, E_TILE // E_BLK, hp_block, 0)
        plsc.subcore_barrier()
        pltpu.sync_copy(acc_sh.at[pl.ds(z0, 640)],
                        acc_hbm.at[pl.ds(h * ND + z0, 640)])
        plsc.subcore_barrier()
        return 0

    lax.fori_loop(0, 4, head_pass, 0)


def _edge_phase(xwh, logits, srcp, dstp, z1):
    mesh = plsc.VectorSubcoreMesh(core_axis_name="c", subcore_axis_name="s")
    xwf = xwh.reshape(H * N, FM)
    lg_pad = jnp.zeros((NR, FM), jnp.float32).at[:N].set(logits)

    p0 = pl.kernel(
        _p0_body,
        out_type=[
            jax.ShapeDtypeStruct((2 * ND, FM), jnp.float32),
            jax.ShapeDtypeStruct((EPC, FM), jnp.float32),
        ],
        mesh=mesh,
        scratch_types=[
            pltpu.VMEM((E_BLK,), jnp.int32),
            pltpu.VMEM((E_BLK,), jnp.int32),
            pltpu.VMEM((16, FM), jnp.float32),
            pltpu.VMEM((16, FM), jnp.float32),
            pltpu.VMEM((16, FM), jnp.float32),
            pltpu.VMEM((E_BLK // 8, FM), jnp.float32),
            pltpu.VMEM_SHARED((NR, FM), jnp.float32),
            pltpu.SemaphoreType.DMA,
            pltpu.SemaphoreType.DMA,
        ],
    )
    den2, ex = p0(lg_pad, srcp, dstp, z1)

    hp = pl.kernel(
        _hp_body,
        out_type=jax.ShapeDtypeStruct((H * ND, FM), jnp.float32),
        mesh=mesh,
        scratch_types=[
            pltpu.VMEM((E_BLK,), jnp.int32),
            pltpu.VMEM((E_BLK,), jnp.int32),
            pltpu.VMEM((E_BLK // 8, FM), jnp.float32),
            pltpu.VMEM((64, FM), jnp.float32),
            pltpu.VMEM((64, FM), jnp.float32),
            pltpu.VMEM((2, 64), jnp.int32),
            pltpu.VMEM((2, 64), jnp.int32),
            pltpu.VMEM_SHARED((NR, FM), jnp.float32),
            pltpu.SemaphoreType.DMA,
            pltpu.SemaphoreType.DMA,
        ],
    )
    acc = hp(xwf, srcp, dstp, ex, z1)
    return (acc.reshape(H, ND, FM)[:, :N],
            den2.reshape(2, ND, FM)[:, :N])


def kernel(x, edge_index, W1, a_src1, a_dst1, b1, W2, a_src2, a_dst2, b2,
           conv_w, conv_b):
    loops = jnp.arange(N, dtype=edge_index.dtype)
    pad = jnp.full((EP - E - N,), DUMP, dtype=edge_index.dtype)
    srcp = jnp.concatenate([edge_index[0], loops, jnp.zeros_like(pad)])
    dstp = jnp.concatenate([edge_index[1], loops, pad])
    z1 = jnp.zeros((NR, FM), jnp.float32)

    xwh1, logits1 = _stage_a(x, W1, a_src1, a_dst1)
    acc1, den1 = _edge_phase(xwh1, logits1, srcp, dstp, z1)
    x1 = _stage_c(acc1, den1, b1)

    xwh2, logits2 = _stage_a(x1, W2, a_src2, a_dst2)
    acc2, den2 = _edge_phase(xwh2, logits2, srcp, dstp, z1)
    x2 = _stage_c(acc2, den2, b2)

    wc = conv_w.reshape(OUT_CH, 2 * FM).T
    xo = _stage_e1(x1, x2, wc[:FM], wc[FM:], conv_b)
    circ = xo[:N_CIRC]
    mirna = xo[N_CIRC:]
    p = _stage_e2(circ, mirna)
    return p, circ, mirna


# speedup vs baseline: 7.0196x; 1.0736x over previous
import jax
import jax.numpy as jnp
from jax import lax
from jax.experimental import pallas as pl
from jax.experimental.pallas import tpu as pltpu
from jax.experimental.pallas import tpu_sc as plsc

N = 10000
FM = 128
H = 8
E = 320000
OUT_CH = 128
N_CIRC = 504

_BLK = 1000

EP = 348160
E_TILE = EP // 16
E_W = EP // 32
E_BLK = 640
NR = 10256
ND = 10240
DUMP = 10240
EPC = EP // 8

_GDN = lax.GatherDimensionNumbers(
    offset_dims=(), collapsed_slice_dims=(0,), start_index_map=(0,))


def _bcast16(v, i):
    idx = jnp.full((16,), i, jnp.int32)
    return lax.gather(v, idx[:, None], _GDN, slice_sizes=(1,),
                      mode=lax.GatherScatterMode.PROMISE_IN_BOUNDS)


def _stage_a_body(x_ref, w_ref, asrc_ref, adst_ref, xw_ref, logits_ref):
    xw = jnp.dot(x_ref[...], w_ref[...], preferred_element_type=jnp.float32)
    xw3 = xw.reshape(_BLK, H, FM)
    xw_ref[...] = xw3.transpose(1, 0, 2)
    als = (xw3 * asrc_ref[...][None]).sum(-1)
    ald = (xw3 * adst_ref[...][None]).sum(-1)
    logits_ref[...] = jnp.concatenate(
        [als, ald, jnp.zeros((_BLK, FM - 2 * H), jnp.float32)], axis=1)


def _stage_a(x, W, a_src, a_dst):
    grid = N // _BLK
    return pl.pallas_call(
        _stage_a_body,
        grid=(grid,),
        in_specs=[
            pl.BlockSpec((_BLK, FM), lambda i: (i, 0)),
            pl.BlockSpec((FM, H * FM), lambda i: (0, 0)),
            pl.BlockSpec((H, FM), lambda i: (0, 0)),
            pl.BlockSpec((H, FM), lambda i: (0, 0)),
        ],
        out_specs=[
            pl.BlockSpec((H, _BLK, FM), lambda i: (0, i, 0)),
            pl.BlockSpec((_BLK, FM), lambda i: (i, 0)),
        ],
        out_shape=[
            jax.ShapeDtypeStruct((H, N, FM), jnp.float32),
            jax.ShapeDtypeStruct((N, FM), jnp.float32),
        ],
    )(x, W, a_src, a_dst)


def _stage_c_body(acc_ref, den_ref, b_ref, out_ref):
    d2 = den_ref[...]
    den = d2[0, :, :H] + d2[1, :, :H] + 1e-16
    acc = acc_ref[...]
    s = jnp.zeros((_BLK, FM), jnp.float32)
    for h in range(H):
        s += acc[h] / den[:, h][:, None]
    out = s * (1.0 / H) + b_ref[...][None, :]
    out_ref[...] = jnp.maximum(out, 0.0)


def _stage_c(acc, denom, b):
    grid = N // _BLK
    return pl.pallas_call(
        _stage_c_body,
        grid=(grid,),
        in_specs=[
            pl.BlockSpec((H, _BLK, FM), lambda i: (0, i, 0)),
            pl.BlockSpec((2, _BLK, FM), lambda i: (0, i, 0)),
            pl.BlockSpec((FM,), lambda i: (0,)),
        ],
        out_specs=pl.BlockSpec((_BLK, FM), lambda i: (i, 0)),
        out_shape=jax.ShapeDtypeStruct((N, FM), jnp.float32),
    )(acc, denom, b)


def _stage_e1_body(x1_ref, x2_ref, wc1_ref, wc2_ref, cb_ref, xo_ref):
    xo = jnp.dot(x1_ref[...], wc1_ref[...], preferred_element_type=jnp.float32)
    xo += jnp.dot(x2_ref[...], wc2_ref[...], preferred_element_type=jnp.float32)
    xo_ref[...] = xo + cb_ref[...][None, :]


def _stage_e1(x1, x2, wc1, wc2, conv_b):
    grid = N // _BLK
    return pl.pallas_call(
        _stage_e1_body,
        grid=(grid,),
        in_specs=[
            pl.BlockSpec((_BLK, FM), lambda i: (i, 0)),
            pl.BlockSpec((_BLK, FM), lambda i: (i, 0)),
            pl.BlockSpec((FM, OUT_CH), lambda i: (0, 0)),
            pl.BlockSpec((FM, OUT_CH), lambda i: (0, 0)),
            pl.BlockSpec((OUT_CH,), lambda i: (0,)),
        ],
        out_specs=pl.BlockSpec((_BLK, OUT_CH), lambda i: (i, 0)),
        out_shape=jax.ShapeDtypeStruct((N, OUT_CH), jnp.float32),
    )(x1, x2, wc1, wc2, conv_b)


def _stage_e2_body(circ_ref, mirna_ref, p_ref):
    p_ref[...] = jax.lax.dot_general(
        circ_ref[...], mirna_ref[...],
        (((1,), (1,)), ((), ())),
        preferred_element_type=jnp.float32)


def _stage_e2(circ, mirna):
    return pl.pallas_call(
        _stage_e2_body,
        in_specs=[
            pl.BlockSpec((N_CIRC, OUT_CH), lambda: (0, 0)),
            pl.BlockSpec((N - N_CIRC, OUT_CH), lambda: (0, 0)),
        ],
        out_specs=pl.BlockSpec((N_CIRC, N - N_CIRC), lambda: (0, 0)),
        out_shape=jax.ShapeDtypeStruct((N_CIRC, N - N_CIRC), jnp.float32),
    )(circ, mirna)


def _p0_body(lg_hbm, src_hbm, dst_hbm, z_hbm,
             den_hbm, ex_hbm,
             src_blk, dst_blk, lrow, lrow2, exmat, exc,
             den_sh, sem1, sem2):
    core = lax.axis_index("c")
    sid = lax.axis_index("s")
    lane = lax.iota(jnp.int32, 16)
    perm8 = (lane + 8) & 15

    z0 = sid * 640
    pltpu.sync_copy(z_hbm.at[pl.ds(z0, 640)], den_sh.at[pl.ds(z0, 640)])

    @pl.when(sid == 0)
    def _ztail():
        pltpu.sync_copy(z_hbm.at[pl.ds(ND, NR - ND)],
                        den_sh.at[pl.ds(ND, NR - ND)])

    pltpu.sync_copy(z_hbm.at[pl.ds(0, 16)], exmat)
    plsc.subcore_barrier()

    wbase = (core * 16 + sid) * E_W

    def p0_block(b, _):
        off = wbase + b * E_BLK
        pltpu.sync_copy(src_hbm.at[pl.ds(off, E_BLK)], src_blk)
        pltpu.sync_copy(dst_hbm.at[pl.ds(off, E_BLK)], dst_blk)

        def p0_batch(j, _):
            sl = pl.ds(j * 16, 16)
            g1 = pltpu.async_copy(lg_hbm.at[src_blk.at[sl]], lrow, sem1)
            g2 = pltpu.async_copy(lg_hbm.at[dst_blk.at[sl]], lrow2, sem2)
            g1.wait()
            g2.wait()

            def p0_edge(e, _):
                a = lrow[e, pl.ds(0, 16)]
                b2 = lax.gather(lrow2[e, pl.ds(0, 16)], perm8[:, None],
                                _GDN, slice_sizes=(1,),
                                mode=lax.GatherScatterMode.PROMISE_IN_BOUNDS)
                sv = a + b2
                alpha = jnp.where(sv >= 0, sv, 0.2 * sv)
                ev = jnp.exp(alpha)
                exmat[e, pl.ds(0, 16)] = ev
                ei = j * 16 + e
                exc[ei >> 3, pl.ds((ei & 7) * 16, 16)] = ev
                return 0

            lax.fori_loop(0, 16, p0_edge, 0)
            dstv = dst_blk[sl]
            pltpu.async_copy(exmat, den_sh.at[dstv], sem1,
                             add=True).wait()
            return 0

        lax.fori_loop(0, E_BLK // 16, p0_batch, 0)
        exoff = pl.multiple_of(off // 8, 8)
        pltpu.sync_copy(exc, ex_hbm.at[pl.ds(exoff, E_BLK // 8)])
        return 0

    lax.fori_loop(0, E_W // E_BLK, p0_block, 0)
    plsc.subcore_barrier()
    pltpu.sync_copy(den_sh.at[pl.ds(z0, 640)],
                    den_hbm.at[pl.ds(core * ND + z0, 640)])


def _hp_body(xwf_hbm, src_hbm, dst_hbm, ex_hbm, z_hbm,
             acc_hbm,
             src_blk, dst_blk, ex_blk, xr0, xr1, xr2, xr3, idxb, dstb,
             acc_sh, sem1, sem2):
    core = lax.axis_index("c")
    sid = lax.axis_index("s")

    z0 = sid * 640
    tbase = sid * E_TILE
    NB = E_BLK // 64

    def head_pass(hl, _):
        h = core * 4 + hl
        pltpu.sync_copy(z_hbm.at[pl.ds(z0, 640)], acc_sh.at[pl.ds(z0, 640)])

        @pl.when(sid == 0)
        def _ztail():
            pltpu.sync_copy(z_hbm.at[pl.ds(ND, NR - ND)],
                            acc_sh.at[pl.ds(ND, NR - ND)])

        plsc.subcore_barrier()

        def hp_block(b, _):
            off = tbase + b * E_BLK
            pltpu.sync_copy(src_hbm.at[pl.ds(off, E_BLK)], src_blk)
            pltpu.sync_copy(dst_hbm.at[pl.ds(off, E_BLK)], dst_blk)
            exoff = pl.multiple_of(off // 8, 8)
            pltpu.sync_copy(ex_hbm.at[pl.ds(exoff, E_BLK // 8)], ex_blk)

            xr = (xr0, xr1, xr2, xr3)

            def fill(j):
                p = j & 3
                for k in range(4):
                    slk = pl.ds(j * 64 + k * 16, 16)
                    idxb[p, pl.ds(k * 16, 16)] = src_blk[slk] + h * N
                    dstb[p, pl.ds(k * 16, 16)] = dst_blk[slk]

            def gather(j):
                p = j & 3
                return pltpu.async_copy(xwf_hbm.at[idxb.at[p]], xr[p], sem1)

            fill(0)
            gq = {0: gather(0)}
            fill(1)
            gq[1] = gather(1)
            sq = {}
            for j in range(NB):
                p = j & 3
                gq.pop(j).wait()
                if j + 2 < NB:
                    if j - 2 >= 0:
                        sq.pop(j - 2).wait()
                    fill(j + 2)
                    gq[j + 2] = gather(j + 2)

                xrp = xr[p]

                def hp_edge(e, _, _j=j, _xrp=xrp):
                    rowv = ex_blk[_j * 8 + (e >> 3), pl.ds((e & 7) * 16, 16)]
                    exh = _bcast16(rowv, h)
                    for q in range(FM // 16):
                        _xrp[e, pl.ds(q * 16, 16)] = (
                            _xrp[e, pl.ds(q * 16, 16)] * exh)
                    return 0

                lax.fori_loop(0, 64, hp_edge, 0)
                sq[j] = pltpu.async_copy(xrp, acc_sh.at[dstb.at[p]], sem2,
                                         add=True)
            for j in sorted(sq):
                sq.pop(j).wait()
            return 0

        lax.fori_loop(0, E_TILE // E_BLK, hp_block, 0)
        plsc.subcore_barrier()
        pltpu.sync_copy(acc_sh.at[pl.ds(z0, 640)],
                        acc_hbm.at[pl.ds(h * ND + z0, 640)])
        plsc.subcore_barrier()
        return 0

    lax.fori_loop(0, 4, head_pass, 0)


def _edge_phase(xwh, logits, srcp, dstp, z1):
    mesh = plsc.VectorSubcoreMesh(core_axis_name="c", subcore_axis_name="s")
    xwf = xwh.reshape(H * N, FM)
    lg_pad = jnp.zeros((NR, FM), jnp.float32).at[:N].set(logits)

    p0 = pl.kernel(
        _p0_body,
        out_type=[
            jax.ShapeDtypeStruct((2 * ND, FM), jnp.float32),
            jax.ShapeDtypeStruct((EPC, FM), jnp.float32),
        ],
        mesh=mesh,
        scratch_types=[
            pltpu.VMEM((E_BLK,), jnp.int32),
            pltpu.VMEM((E_BLK,), jnp.int32),
            pltpu.VMEM((16, FM), jnp.float32),
            pltpu.VMEM((16, FM), jnp.float32),
            pltpu.VMEM((16, FM), jnp.float32),
            pltpu.VMEM((E_BLK // 8, FM), jnp.float32),
            pltpu.VMEM_SHARED((NR, FM), jnp.float32),
            pltpu.SemaphoreType.DMA,
            pltpu.SemaphoreType.DMA,
        ],
    )
    den2, ex = p0(lg_pad, srcp, dstp, z1)

    hp = pl.kernel(
        _hp_body,
        out_type=jax.ShapeDtypeStruct((H * ND, FM), jnp.float32),
        mesh=mesh,
        scratch_types=[
            pltpu.VMEM((E_BLK,), jnp.int32),
            pltpu.VMEM((E_BLK,), jnp.int32),
            pltpu.VMEM((E_BLK // 8, FM), jnp.float32),
            pltpu.VMEM((64, FM), jnp.float32),
            pltpu.VMEM((64, FM), jnp.float32),
            pltpu.VMEM((64, FM), jnp.float32),
            pltpu.VMEM((64, FM), jnp.float32),
            pltpu.VMEM((4, 64), jnp.int32),
            pltpu.VMEM((4, 64), jnp.int32),
            pltpu.VMEM_SHARED((NR, FM), jnp.float32),
            pltpu.SemaphoreType.DMA,
            pltpu.SemaphoreType.DMA,
        ],
    )
    acc = hp(xwf, srcp, dstp, ex, z1)
    return (acc.reshape(H, ND, FM)[:, :N],
            den2.reshape(2, ND, FM)[:, :N])


def kernel(x, edge_index, W1, a_src1, a_dst1, b1, W2, a_src2, a_dst2, b2,
           conv_w, conv_b):
    loops = jnp.arange(N, dtype=edge_index.dtype)
    pad = jnp.full((EP - E - N,), DUMP, dtype=edge_index.dtype)
    srcp = jnp.concatenate([edge_index[0], loops, jnp.zeros_like(pad)])
    dstp = jnp.concatenate([edge_index[1], loops, pad])
    z1 = jnp.zeros((NR, FM), jnp.float32)

    xwh1, logits1 = _stage_a(x, W1, a_src1, a_dst1)
    acc1, den1 = _edge_phase(xwh1, logits1, srcp, dstp, z1)
    x1 = _stage_c(acc1, den1, b1)

    xwh2, logits2 = _stage_a(x1, W2, a_src2, a_dst2)
    acc2, den2 = _edge_phase(xwh2, logits2, srcp, dstp, z1)
    x2 = _stage_c(acc2, den2, b2)

    wc = conv_w.reshape(OUT_CH, 2 * FM).T
    xo = _stage_e1(x1, x2, wc[:FM], wc[FM:], conv_b)
    circ = xo[:N_CIRC]
    mirna = xo[N_CIRC:]
    p = _stage_e2(circ, mirna)
    return p, circ, mirna


# speedup vs baseline: 7.1428x; 1.0175x over previous
import jax
import jax.numpy as jnp
from jax import lax
from jax.experimental import pallas as pl
from jax.experimental.pallas import tpu as pltpu
from jax.experimental.pallas import tpu_sc as plsc

N = 10000
FM = 128
H = 8
E = 320000
OUT_CH = 128
N_CIRC = 504

_BLK = 1000

EP = 348160
E_TILE = EP // 16
E_W = EP // 32
E_BLK = 640
NR = 10256
ND = 10240
DUMP = 10240
EPC = EP // 8

_GDN = lax.GatherDimensionNumbers(
    offset_dims=(), collapsed_slice_dims=(0,), start_index_map=(0,))


def _bcast16(v, i):
    idx = jnp.full((16,), i, jnp.int32)
    return lax.gather(v, idx[:, None], _GDN, slice_sizes=(1,),
                      mode=lax.GatherScatterMode.PROMISE_IN_BOUNDS)


def _stage_a_body(x_ref, w_ref, asrc_ref, adst_ref, xw_ref, logits_ref):
    xw = jnp.dot(x_ref[...], w_ref[...], preferred_element_type=jnp.float32)
    xw3 = xw.reshape(_BLK, H, FM)
    xw_ref[...] = xw3.transpose(1, 0, 2)
    als = (xw3 * asrc_ref[...][None]).sum(-1)
    ald = (xw3 * adst_ref[...][None]).sum(-1)
    logits_ref[...] = jnp.concatenate(
        [als, ald, jnp.zeros((_BLK, FM - 2 * H), jnp.float32)], axis=1)


def _stage_a(x, W, a_src, a_dst):
    grid = N // _BLK
    return pl.pallas_call(
        _stage_a_body,
        grid=(grid,),
        in_specs=[
            pl.BlockSpec((_BLK, FM), lambda i: (i, 0)),
            pl.BlockSpec((FM, H * FM), lambda i: (0, 0)),
            pl.BlockSpec((H, FM), lambda i: (0, 0)),
            pl.BlockSpec((H, FM), lambda i: (0, 0)),
        ],
        out_specs=[
            pl.BlockSpec((H, _BLK, FM), lambda i: (0, i, 0)),
            pl.BlockSpec((_BLK, FM), lambda i: (i, 0)),
        ],
        out_shape=[
            jax.ShapeDtypeStruct((H, N, FM), jnp.float32),
            jax.ShapeDtypeStruct((N, FM), jnp.float32),
        ],
    )(x, W, a_src, a_dst)


def _stage_c_body(acc_ref, den_ref, b_ref, out_ref):
    d2 = den_ref[...]
    den = d2[0, :, :H] + d2[1, :, :H] + 1e-16
    acc = acc_ref[...]
    s = jnp.zeros((_BLK, FM), jnp.float32)
    for h in range(H):
        s += acc[h] / den[:, h][:, None]
    out = s * (1.0 / H) + b_ref[...][None, :]
    out_ref[...] = jnp.maximum(out, 0.0)


def _stage_c(acc, denom, b):
    grid = N // _BLK
    return pl.pallas_call(
        _stage_c_body,
        grid=(grid,),
        in_specs=[
            pl.BlockSpec((H, _BLK, FM), lambda i: (0, i, 0)),
            pl.BlockSpec((2, _BLK, FM), lambda i: (0, i, 0)),
            pl.BlockSpec((FM,), lambda i: (0,)),
        ],
        out_specs=pl.BlockSpec((_BLK, FM), lambda i: (i, 0)),
        out_shape=jax.ShapeDtypeStruct((N, FM), jnp.float32),
    )(acc, denom, b)


def _stage_e1_body(x1_ref, x2_ref, wc1_ref, wc2_ref, cb_ref, xo_ref):
    xo = jnp.dot(x1_ref[...], wc1_ref[...], preferred_element_type=jnp.float32)
    xo += jnp.dot(x2_ref[...], wc2_ref[...], preferred_element_type=jnp.float32)
    xo_ref[...] = xo + cb_ref[...][None, :]


def _stage_e1(x1, x2, wc1, wc2, conv_b):
    grid = N // _BLK
    return pl.pallas_call(
        _stage_e1_body,
        grid=(grid,),
        in_specs=[
            pl.BlockSpec((_BLK, FM), lambda i: (i, 0)),
            pl.BlockSpec((_BLK, FM), lambda i: (i, 0)),
            pl.BlockSpec((FM, OUT_CH), lambda i: (0, 0)),
            pl.BlockSpec((FM, OUT_CH), lambda i: (0, 0)),
            pl.BlockSpec((OUT_CH,), lambda i: (0,)),
        ],
        out_specs=pl.BlockSpec((_BLK, OUT_CH), lambda i: (i, 0)),
        out_shape=jax.ShapeDtypeStruct((N, OUT_CH), jnp.float32),
    )(x1, x2, wc1, wc2, conv_b)


def _stage_e2_body(circ_ref, mirna_ref, p_ref):
    p_ref[...] = jax.lax.dot_general(
        circ_ref[...], mirna_ref[...],
        (((1,), (1,)), ((), ())),
        preferred_element_type=jnp.float32)


def _stage_e2(circ, mirna):
    return pl.pallas_call(
        _stage_e2_body,
        in_specs=[
            pl.BlockSpec((N_CIRC, OUT_CH), lambda: (0, 0)),
            pl.BlockSpec((N - N_CIRC, OUT_CH), lambda: (0, 0)),
        ],
        out_specs=pl.BlockSpec((N_CIRC, N - N_CIRC), lambda: (0, 0)),
        out_shape=jax.ShapeDtypeStruct((N_CIRC, N - N_CIRC), jnp.float32),
    )(circ, mirna)


def _p0_body(lg_hbm, src_hbm, dst_hbm, z_hbm,
             den_hbm, ex_hbm,
             src_blk, dst_blk, lrow, lrow2, exmat, exc, dstb,
             den_sh, sem1, sem2):
    core = lax.axis_index("c")
    sid = lax.axis_index("s")
    lane = lax.iota(jnp.int32, 16)
    perm8 = (lane + 8) & 15

    z0 = sid * 640
    pltpu.sync_copy(z_hbm.at[pl.ds(z0, 640)], den_sh.at[pl.ds(z0, 640)])

    @pl.when(sid == 0)
    def _ztail():
        pltpu.sync_copy(z_hbm.at[pl.ds(ND, NR - ND)],
                        den_sh.at[pl.ds(ND, NR - ND)])

    pltpu.sync_copy(z_hbm.at[pl.ds(0, 64)], exmat)
    plsc.subcore_barrier()

    wbase = (core * 16 + sid) * E_W

    def p0_block(b, _):
        off = wbase + b * E_BLK
        pltpu.sync_copy(src_hbm.at[pl.ds(off, E_BLK)], src_blk)
        pltpu.sync_copy(dst_hbm.at[pl.ds(off, E_BLK)], dst_blk)

        def p0_batch(j, _):
            sl = pl.ds(j * 64, 64)
            for k in range(4):
                slk = pl.ds(j * 64 + k * 16, 16)
                dstb[pl.ds(k * 16, 16)] = dst_blk[slk]
            g1 = pltpu.async_copy(lg_hbm.at[src_blk.at[sl]], lrow, sem1)
            g2 = pltpu.async_copy(lg_hbm.at[dst_blk.at[sl]], lrow2, sem2)
            g1.wait()
            g2.wait()

            def p0_edge(e, _):
                a = lrow[e, pl.ds(0, 16)]
                b2 = lax.gather(lrow2[e, pl.ds(0, 16)], perm8[:, None],
                                _GDN, slice_sizes=(1,),
                                mode=lax.GatherScatterMode.PROMISE_IN_BOUNDS)
                sv = a + b2
                alpha = jnp.where(sv >= 0, sv, 0.2 * sv)
                ev = jnp.exp(alpha)
                exmat[e, pl.ds(0, 16)] = ev
                ei = j * 64 + e
                exc[ei >> 3, pl.ds((ei & 7) * 16, 16)] = ev
                return 0

            lax.fori_loop(0, 64, p0_edge, 0)
            pltpu.async_copy(exmat, den_sh.at[dstb], sem1,
                             add=True).wait()
            return 0

        lax.fori_loop(0, E_BLK // 64, p0_batch, 0)
        exoff = pl.multiple_of(off // 8, 8)
        pltpu.sync_copy(exc, ex_hbm.at[pl.ds(exoff, E_BLK // 8)])
        return 0

    lax.fori_loop(0, E_W // E_BLK, p0_block, 0)
    plsc.subcore_barrier()
    pltpu.sync_copy(den_sh.at[pl.ds(z0, 640)],
                    den_hbm.at[pl.ds(core * ND + z0, 640)])


def _hp_body(xwf_hbm, src_hbm, dst_hbm, ex_hbm, z_hbm,
             acc_hbm,
             src_blk, dst_blk, ex_blk, xr0, xr1, xr2, xr3, idxb, dstb,
             acc_sh, sem1, sem2):
    core = lax.axis_index("c")
    sid = lax.axis_index("s")

    z0 = sid * 640
    tbase = sid * E_TILE
    NB = E_BLK // 64

    def head_pass(hl, _):
        h = core * 4 + hl
        pltpu.sync_copy(z_hbm.at[pl.ds(z0, 640)], acc_sh.at[pl.ds(z0, 640)])

        @pl.when(sid == 0)
        def _ztail():
            pltpu.sync_copy(z_hbm.at[pl.ds(ND, NR - ND)],
                            acc_sh.at[pl.ds(ND, NR - ND)])

        plsc.subcore_barrier()

        def hp_block(b, _):
            off = tbase + b * E_BLK
            pltpu.sync_copy(src_hbm.at[pl.ds(off, E_BLK)], src_blk)
            pltpu.sync_copy(dst_hbm.at[pl.ds(off, E_BLK)], dst_blk)
            exoff = pl.multiple_of(off // 8, 8)
            pltpu.sync_copy(ex_hbm.at[pl.ds(exoff, E_BLK // 8)], ex_blk)

            xr = (xr0, xr1, xr2, xr3)

            def fill(j):
                p = j & 3
                for k in range(4):
                    slk = pl.ds(j * 64 + k * 16, 16)
                    idxb[p, pl.ds(k * 16, 16)] = src_blk[slk] + h * N
                    dstb[p, pl.ds(k * 16, 16)] = dst_blk[slk]

            def gather(j):
                p = j & 3
                return pltpu.async_copy(xwf_hbm.at[idxb.at[p]], xr[p], sem1)

            fill(0)
            gq = {0: gather(0)}
            fill(1)
            gq[1] = gather(1)
            sq = {}
            for j in range(NB):
                p = j & 3
                gq.pop(j).wait()
                if j + 2 < NB:
                    if j - 2 >= 0:
                        sq.pop(j - 2).wait()
                    fill(j + 2)
                    gq[j + 2] = gather(j + 2)

                xrp = xr[p]

                def hp_edge(e, _, _j=j, _xrp=xrp):
                    rowv = ex_blk[_j * 8 + (e >> 3), pl.ds((e & 7) * 16, 16)]
                    exh = _bcast16(rowv, h)
                    for q in range(FM // 16):
                        _xrp[e, pl.ds(q * 16, 16)] = (
                            _xrp[e, pl.ds(q * 16, 16)] * exh)
                    return 0

                lax.fori_loop(0, 64, hp_edge, 0, unroll=4)
                sq[j] = pltpu.async_copy(xrp, acc_sh.at[dstb.at[p]], sem2,
                                         add=True)
            for j in sorted(sq):
                sq.pop(j).wait()
            return 0

        lax.fori_loop(0, E_TILE // E_BLK, hp_block, 0)
        plsc.subcore_barrier()
        pltpu.sync_copy(acc_sh.at[pl.ds(z0, 640)],
                        acc_hbm.at[pl.ds(h * ND + z0, 640)])
        plsc.subcore_barrier()
        return 0

    lax.fori_loop(0, 4, head_pass, 0)


def _edge_phase(xwh, logits, srcp, dstp, z1):
    mesh = plsc.VectorSubcoreMesh(core_axis_name="c", subcore_axis_name="s")
    xwf = xwh.reshape(H * N, FM)
    lg_pad = jnp.zeros((NR, FM), jnp.float32).at[:N].set(logits)

    p0 = pl.kernel(
        _p0_body,
        out_type=[
            jax.ShapeDtypeStruct((2 * ND, FM), jnp.float32),
            jax.ShapeDtypeStruct((EPC, FM), jnp.float32),
        ],
        mesh=mesh,
        scratch_types=[
            pltpu.VMEM((E_BLK,), jnp.int32),
            pltpu.VMEM((E_BLK,), jnp.int32),
            pltpu.VMEM((64, FM), jnp.float32),
            pltpu.VMEM((64, FM), jnp.float32),
            pltpu.VMEM((64, FM), jnp.float32),
            pltpu.VMEM((E_BLK // 8, FM), jnp.float32),
            pltpu.VMEM((64,), jnp.int32),
            pltpu.VMEM_SHARED((NR, FM), jnp.float32),
            pltpu.SemaphoreType.DMA,
            pltpu.SemaphoreType.DMA,
        ],
    )
    den2, ex = p0(lg_pad, srcp, dstp, z1)

    hp = pl.kernel(
        _hp_body,
        out_type=jax.ShapeDtypeStruct((H * ND, FM), jnp.float32),
        mesh=mesh,
        scratch_types=[
            pltpu.VMEM((E_BLK,), jnp.int32),
            pltpu.VMEM((E_BLK,), jnp.int32),
            pltpu.VMEM((E_BLK // 8, FM), jnp.float32),
            pltpu.VMEM((64, FM), jnp.float32),
            pltpu.VMEM((64, FM), jnp.float32),
            pltpu.VMEM((64, FM), jnp.float32),
            pltpu.VMEM((64, FM), jnp.float32),
            pltpu.VMEM((4, 64), jnp.int32),
            pltpu.VMEM((4, 64), jnp.int32),
            pltpu.VMEM_SHARED((NR, FM), jnp.float32),
            pltpu.SemaphoreType.DMA,
            pltpu.SemaphoreType.DMA,
        ],
    )
    acc = hp(xwf, srcp, dstp, ex, z1)
    return (acc.reshape(H, ND, FM)[:, :N],
            den2.reshape(2, ND, FM)[:, :N])


def kernel(x, edge_index, W1, a_src1, a_dst1, b1, W2, a_src2, a_dst2, b2,
           conv_w, conv_b):
    loops = jnp.arange(N, dtype=edge_index.dtype)
    pad = jnp.full((EP - E - N,), DUMP, dtype=edge_index.dtype)
    srcp = jnp.concatenate([edge_index[0], loops, jnp.zeros_like(pad)])
    dstp = jnp.concatenate([edge_index[1], loops, pad])
    z1 = jnp.zeros((NR, FM), jnp.float32)

    xwh1, logits1 = _stage_a(x, W1, a_src1, a_dst1)
    acc1, den1 = _edge_phase(xwh1, logits1, srcp, dstp, z1)
    x1 = _stage_c(acc1, den1, b1)

    xwh2, logits2 = _stage_a(x1, W2, a_src2, a_dst2)
    acc2, den2 = _edge_phase(xwh2, logits2, srcp, dstp, z1)
    x2 = _stage_c(acc2, den2, b2)

    wc = conv_w.reshape(OUT_CH, 2 * FM).T
    xo = _stage_e1(x1, x2, wc[:FM], wc[FM:], conv_b)
    circ = xo[:N_CIRC]
    mirna = xo[N_CIRC:]
    p = _stage_e2(circ, mirna)
    return p, circ, mirna
